# Initial kernel scaffold; baseline (speedup 1.0000x reference)
import jax
import jax.numpy as jnp
from jax.experimental import pallas as pl


def kernel(x, table, pos_encoding):
    raise NotImplementedError("write your pallas kernel here")


# speedup vs baseline: 4.6582x; 4.6582x over previous
import functools
import math

import jax
import jax.numpy as jnp
from jax import lax
from jax.experimental import pallas as pl
from jax.experimental.pallas import tpu as pltpu
from jax.experimental.pallas import tpu_sc as plsc

D = 32
SEQ = 200
BATCH = 4096
NC = 2
NS = 16
NW = NC * NS
LANES = 16
ROWS_TOTAL = BATCH * SEQ
ROWS_PER_W = ROWS_TOTAL // NW
CHUNK = 1600
NCHUNKS = ROWS_PER_W // CHUNK
REPS = CHUNK // SEQ
SCALE = math.sqrt(float(D))

_mesh = plsc.VectorSubcoreMesh(core_axis_name="c", subcore_axis_name="s")


@functools.partial(
    pl.kernel,
    out_type=jax.ShapeDtypeStruct((ROWS_TOTAL, D), jnp.float32),
    mesh=_mesh,
    compiler_params=pltpu.CompilerParams(use_tc_tiling_on_sc=False),
    scratch_types=[
        pltpu.VMEM((CHUNK,), jnp.int32),
        pltpu.VMEM((CHUNK,), jnp.int32),
        pltpu.VMEM((CHUNK, D), jnp.float32),
        pltpu.VMEM((CHUNK, D), jnp.float32),
        pltpu.VMEM((SEQ, D), jnp.float32),
        pltpu.SemaphoreType.DMA,
        pltpu.SemaphoreType.DMA,
        pltpu.SemaphoreType.DMA,
        pltpu.SemaphoreType.DMA,
    ],
)
def _sc_embed(x_hbm, table_hbm, pos_hbm, out_hbm,
              idx0, idx1, rows0, rows1, pos_v,
              gsem0, gsem1, osem0, osem1):
    wid = lax.axis_index("s") * NC + lax.axis_index("c")
    row0 = wid * ROWS_PER_W

    idx = (idx0, idx1)
    rows = (rows0, rows1)
    gsem = (gsem0, gsem1)
    osem = (osem0, osem1)

    pltpu.sync_copy(pos_hbm.at[pl.ds(0, SEQ)], pos_v)

    def fire(g, buf):
        base = row0 + g * CHUNK
        pltpu.sync_copy(x_hbm.at[pl.ds(base, CHUNK)], idx[buf])
        return pltpu.async_copy(table_hbm.at[idx[buf]], rows[buf], gsem[buf])

    def compute(buf):
        r_ref = rows[buf]

        def l_body(l, carry):
            p0 = pos_v[l, pl.ds(0, LANES)]
            p1 = pos_v[l, pl.ds(LANES, LANES)]
            for rep in range(REPS):
                r = rep * SEQ + l
                r_ref[r, pl.ds(0, LANES)] = r_ref[r, pl.ds(0, LANES)] * SCALE + p0
                r_ref[r, pl.ds(LANES, LANES)] = (
                    r_ref[r, pl.ds(LANES, LANES)] * SCALE + p1)
            return carry

        lax.fori_loop(0, SEQ, l_body, 0)

    hg = {0: fire(0, 0)}
    hout = {}
    for g in range(NCHUNKS):
        buf = g % 2
        if g + 1 < NCHUNKS:
            if g - 1 >= 0:
                hout[g - 1].wait()
            hg[g + 1] = fire(g + 1, 1 - buf)
        hg[g].wait()
        compute(buf)
        base = row0 + g * CHUNK
        hout[g] = pltpu.async_copy(rows[buf], out_hbm.at[pl.ds(base, CHUNK)],
                                   osem[buf])
    hout[NCHUNKS - 2].wait()
    hout[NCHUNKS - 1].wait()


def kernel(x, table, pos_encoding):
    xf = x.reshape(ROWS_TOTAL)
    out = _sc_embed(xf, table, pos_encoding)
    return out.reshape(BATCH, SEQ, D)
